# baseline (speedup 1.0000x reference)
import functools

import jax
import jax.numpy as jnp
from jax import lax
from jax.experimental import pallas as pl
from jax.experimental.pallas import tpu as pltpu
from jax.experimental.pallas import tpu_sc as plsc

NC = 2
NS = 16
NW = NC * NS

HID = 64
LANES = 16
JOBS = HID // NW


def _body(idx_hbm, pos_hbm, emb_hbm, out_hbm,
          row_v, pos_v, idx_v, out_v, isem, osem):
    seq, b_tot = idx_hbm.shape
    vocab = emb_hbm.shape[1]
    wid = lax.axis_index("s") * NC + lax.axis_index("c")

    def start_idx_load(s, slot):
        pltpu.async_copy(idx_hbm.at[s], idx_v.at[slot], isem.at[slot])

    def wait_idx_load(s, slot):
        pltpu.make_async_copy(idx_hbm.at[s], idx_v.at[slot], isem.at[slot]).wait()

    for j in range(JOBS):
        h = wid * JOBS + j
        pltpu.sync_copy(emb_hbm.at[h], row_v)
        pltpu.sync_copy(pos_hbm.at[h], pos_v)

        start_idx_load(0, 0)

        def process(s, slot):
            @pl.when(s + 1 < seq)
            def _():
                start_idx_load(s + 1, 1 - slot)

            wait_idx_load(s, slot)

            @pl.when(s >= 2)
            def _():
                pltpu.make_async_copy(
                    out_v.at[slot], out_hbm.at[s - 2, h], osem.at[slot]
                ).wait()

            sv = jnp.full((LANES,), s, jnp.int32)
            ps = plsc.load_gather(pos_v, [sv])

            def vec_step(i, carry2):
                t16 = idx_v[slot, pl.ds(i * LANES, LANES)]
                vals = plsc.load_gather(row_v, [t16])
                out_v[slot, pl.ds(i * LANES, LANES)] = vals + ps
                return carry2

            lax.fori_loop(0, b_tot // LANES, vec_step, 0, unroll=8)

            pltpu.async_copy(out_v.at[slot], out_hbm.at[s, h], osem.at[slot])

        def pair_step(p, carry):
            process(2 * p, 0)
            process(2 * p + 1, 1)
            return carry

        lax.fori_loop(0, seq // 2, pair_step, 0)

        for s in (seq - 2, seq - 1):
            pltpu.make_async_copy(
                out_v.at[s % 2], out_hbm.at[s, h], osem.at[s % 2]
            ).wait()


def kernel(input_x, emb_table, pos_table):
    batch, seq_len = input_x.shape
    vocab, hid = emb_table.shape
    idx_t = input_x.T.astype(jnp.int32)
    emb_t = emb_table.T
    pos_t = pos_table[:seq_len].T

    mesh = plsc.VectorSubcoreMesh(
        core_axis_name="c", subcore_axis_name="s", num_cores=NC, num_subcores=NS
    )
    out = pl.kernel(
        _body,
        out_type=jax.ShapeDtypeStruct((seq_len, hid, batch), jnp.float32),
        mesh=mesh,
        scratch_types=[
            pltpu.VMEM((vocab,), jnp.float32),
            pltpu.VMEM((seq_len,), jnp.float32),
            pltpu.VMEM((2, batch), jnp.int32),
            pltpu.VMEM((2, batch), jnp.float32),
            pltpu.SemaphoreType.DMA((2,)),
            pltpu.SemaphoreType.DMA((2,)),
        ],
        compiler_params=pltpu.CompilerParams(
            use_tc_tiling_on_sc=True, needs_layout_passes=False
        ),
    )(idx_t, pos_t, emb_t)
    return out.transpose(2, 0, 1)


# speedup vs baseline: 2.6067x; 2.6067x over previous
import functools

import jax
import jax.numpy as jnp
from jax import lax
from jax.experimental import pallas as pl
from jax.experimental.pallas import tpu as pltpu
from jax.experimental.pallas import tpu_sc as plsc

NC = 2
NS = 16
NW = NC * NS

HID = 64
LANES = 16
JOBS = HID // NW


def _body(idx_hbm, pos_hbm, emb_hbm, out_hbm,
          row_v, pos_v, idx_v, out_v, isem, osem):
    seq, b_tot = idx_hbm.shape
    vocab = emb_hbm.shape[1]
    wid = lax.axis_index("s") * NC + lax.axis_index("c")

    def start_idx_load(s, slot):
        pltpu.async_copy(idx_hbm.at[s], idx_v.at[slot], isem.at[slot])

    def wait_idx_load(s, slot):
        pltpu.make_async_copy(idx_hbm.at[s], idx_v.at[slot], isem.at[slot]).wait()

    for j in range(JOBS):
        h = wid * JOBS + j
        pltpu.sync_copy(emb_hbm.at[h], row_v)
        pltpu.sync_copy(pos_hbm.at[h], pos_v)

        start_idx_load(0, 0)

        def process(s, slot):
            @pl.when(s + 1 < seq)
            def _():
                start_idx_load(s + 1, 1 - slot)

            wait_idx_load(s, slot)

            @pl.when(s >= 2)
            def _():
                pltpu.make_async_copy(
                    out_v.at[slot], out_hbm.at[s - 2, h], osem.at[slot]
                ).wait()

            sv = jnp.full((LANES,), s, jnp.int32)
            ps = plsc.load_gather(pos_v, [sv])

            GRP = 8

            def vec_step(g, carry2):
                base2 = g * (GRP * LANES)
                t16s = [idx_v[slot, pl.ds(base2 + k * LANES, LANES)]
                        for k in range(GRP)]
                gots = [plsc.load_gather(row_v, [t16]) for t16 in t16s]
                for k in range(GRP):
                    out_v[slot, pl.ds(base2 + k * LANES, LANES)] = gots[k] + ps
                return carry2

            lax.fori_loop(0, b_tot // (GRP * LANES), vec_step, 0)

            pltpu.async_copy(out_v.at[slot], out_hbm.at[s, h], osem.at[slot])

        def pair_step(p, carry):
            process(2 * p, 0)
            process(2 * p + 1, 1)
            return carry

        lax.fori_loop(0, seq // 2, pair_step, 0)

        for s in (seq - 2, seq - 1):
            pltpu.make_async_copy(
                out_v.at[s % 2], out_hbm.at[s, h], osem.at[s % 2]
            ).wait()


def kernel(input_x, emb_table, pos_table):
    batch, seq_len = input_x.shape
    vocab, hid = emb_table.shape
    idx_t = input_x.T.astype(jnp.int32)
    emb_t = emb_table.T
    pos_t = pos_table[:seq_len].T

    mesh = plsc.VectorSubcoreMesh(
        core_axis_name="c", subcore_axis_name="s", num_cores=NC, num_subcores=NS
    )
    out = pl.kernel(
        _body,
        out_type=jax.ShapeDtypeStruct((seq_len, hid, batch), jnp.float32),
        mesh=mesh,
        scratch_types=[
            pltpu.VMEM((vocab,), jnp.float32),
            pltpu.VMEM((seq_len,), jnp.float32),
            pltpu.VMEM((2, batch), jnp.int32),
            pltpu.VMEM((2, batch), jnp.float32),
            pltpu.SemaphoreType.DMA((2,)),
            pltpu.SemaphoreType.DMA((2,)),
        ],
        compiler_params=pltpu.CompilerParams(
            use_tc_tiling_on_sc=True, needs_layout_passes=False
        ),
    )(idx_t, pos_t, emb_t)
    return out.transpose(2, 0, 1)


# speedup vs baseline: 3.0868x; 1.1842x over previous
import functools

import jax
import jax.numpy as jnp
from jax import lax
from jax.experimental import pallas as pl
from jax.experimental.pallas import tpu as pltpu
from jax.experimental.pallas import tpu_sc as plsc

NC = 2
NS = 16
NW = NC * NS

HID = 64
LANES = 16
JOBS = HID // NW


def _body(idx_hbm, pos_hbm, emb_hbm, out_hbm,
          row_v, pos_v, idx_v, out_v, idx_sh, isem, osem):
    seq = idx_hbm.shape[0]
    b_tot = idx_hbm.shape[1] * idx_hbm.shape[2]
    vocab = emb_hbm.shape[1]
    wid = lax.axis_index("s") * NC + lax.axis_index("c")

    sid = lax.axis_index("s")
    for k in range((seq + NS - 1) // NS):
        r = sid + k * NS

        @pl.when(r < seq)
        def _():
            pltpu.sync_copy(idx_hbm.at[r], idx_v.at[0])
            pltpu.sync_copy(idx_v.at[0], idx_sh.at[r])

    plsc.subcore_barrier()

    def start_idx_load(s, slot):
        pltpu.async_copy(idx_sh.at[s], idx_v.at[slot], isem.at[slot])

    def wait_idx_load(s, slot):
        pltpu.make_async_copy(idx_sh.at[s], idx_v.at[slot], isem.at[slot]).wait()

    for j in range(JOBS):
        h = wid * JOBS + j
        pltpu.sync_copy(emb_hbm.at[h], row_v)
        pltpu.sync_copy(pos_hbm.at[h], pos_v)

        start_idx_load(0, 0)

        def process(s, slot):
            @pl.when(s + 1 < seq)
            def _():
                start_idx_load(s + 1, 1 - slot)

            wait_idx_load(s, slot)

            @pl.when(s >= 2)
            def _():
                pltpu.make_async_copy(
                    out_v.at[slot], out_hbm.at[s - 2, h], osem.at[slot]
                ).wait()

            sv = jnp.full((LANES,), s, jnp.int32)
            ps = plsc.load_gather(pos_v, [sv])

            GRP = 8

            def vec_step(g, carry2):
                t16s = [idx_v[slot, g, pl.ds(k * LANES, LANES)]
                        for k in range(GRP)]
                gots = [plsc.load_gather(row_v, [t16]) for t16 in t16s]
                for k in range(GRP):
                    out_v[slot, pl.ds(g * (GRP * LANES) + k * LANES, LANES)] = (
                        gots[k] + ps)
                return carry2

            lax.fori_loop(0, b_tot // (GRP * LANES), vec_step, 0)

            pltpu.async_copy(out_v.at[slot], out_hbm.at[s, h], osem.at[slot])

        def pair_step(p, carry):
            process(2 * p, 0)
            process(2 * p + 1, 1)
            return carry

        lax.fori_loop(0, seq // 2, pair_step, 0)

        for s in (seq - 2, seq - 1):
            pltpu.make_async_copy(
                out_v.at[s % 2], out_hbm.at[s, h], osem.at[s % 2]
            ).wait()


def kernel(input_x, emb_table, pos_table):
    batch, seq_len = input_x.shape
    vocab, hid = emb_table.shape
    idx_t = input_x.T.astype(jnp.int32).reshape(seq_len, batch // 128, 128)
    emb_t = emb_table.T
    pos_t = pos_table[:seq_len].T

    mesh = plsc.VectorSubcoreMesh(
        core_axis_name="c", subcore_axis_name="s", num_cores=NC, num_subcores=NS
    )
    out = pl.kernel(
        _body,
        out_type=jax.ShapeDtypeStruct((seq_len, hid, batch), jnp.float32),
        mesh=mesh,
        scratch_types=[
            pltpu.VMEM((vocab,), jnp.float32),
            pltpu.VMEM((seq_len,), jnp.float32),
            pltpu.VMEM((2, batch // 128, 128), jnp.int32),
            pltpu.VMEM((2, batch), jnp.float32),
            pltpu.VMEM_SHARED((seq_len, batch // 128, 128), jnp.int32),
            pltpu.SemaphoreType.DMA((2,)),
            pltpu.SemaphoreType.DMA((2,)),
        ],
        compiler_params=pltpu.CompilerParams(
            use_tc_tiling_on_sc=True, needs_layout_passes=False
        ),
    )(idx_t, pos_t, emb_t)
    return out.transpose(2, 0, 1)


# speedup vs baseline: 3.1604x; 1.0238x over previous
import functools

import jax
import jax.numpy as jnp
from jax import lax
from jax.experimental import pallas as pl
from jax.experimental.pallas import tpu as pltpu
from jax.experimental.pallas import tpu_sc as plsc

NC = 2
NS = 16
NW = NC * NS

HID = 64
LANES = 16
JOBS = HID // NW


def _body(idx_hbm, pos_hbm, emb_hbm, out_hbm,
          row_v, pos_v, idx_v, out_v, idx_sh, isem, osem):
    seq = idx_hbm.shape[0]
    b_tot = idx_hbm.shape[1] * idx_hbm.shape[2]
    vocab = emb_hbm.shape[1]
    wid = lax.axis_index("s") * NC + lax.axis_index("c")

    sid = lax.axis_index("s")
    for k in range((seq + NS - 1) // NS):
        r = sid + k * NS

        @pl.when(r < seq)
        def _():
            pltpu.sync_copy(idx_hbm.at[r], idx_v.at[0])
            pltpu.sync_copy(idx_v.at[0], idx_sh.at[r])

    plsc.subcore_barrier()

    def start_idx_load(s, slot):
        pltpu.async_copy(idx_sh.at[s], idx_v.at[slot], isem.at[slot])

    def wait_idx_load(s, slot):
        pltpu.make_async_copy(idx_sh.at[s], idx_v.at[slot], isem.at[slot]).wait()

    for j in range(JOBS):
        h = wid * JOBS + j
        pltpu.sync_copy(emb_hbm.at[h], row_v)
        pltpu.sync_copy(pos_hbm.at[h], pos_v)

        start_idx_load(0, 0)

        def process(s, slot):
            @pl.when(s + 1 < seq)
            def _():
                start_idx_load(s + 1, 1 - slot)

            wait_idx_load(s, slot)

            @pl.when(s >= 2)
            def _():
                pltpu.make_async_copy(
                    out_v.at[slot], out_hbm.at[s - 2, h], osem.at[slot]
                ).wait()

            sv = jnp.full((LANES,), s, jnp.int32)
            ps = plsc.load_gather(pos_v, [sv])

            GRP = 16

            def vec_step(g, carry2):
                t16s = [idx_v[slot, 2 * g + (k // 8), pl.ds((k % 8) * LANES, LANES)]
                        for k in range(GRP)]
                gots = [plsc.load_gather(row_v, [t16]) for t16 in t16s]
                for k in range(GRP):
                    out_v[slot, pl.ds(g * (GRP * LANES) + k * LANES, LANES)] = (
                        gots[k] + ps)
                return carry2

            lax.fori_loop(0, b_tot // (GRP * LANES), vec_step, 0)

            pltpu.async_copy(out_v.at[slot], out_hbm.at[s, h], osem.at[slot])

        def pair_step(p, carry):
            process(2 * p, 0)
            process(2 * p + 1, 1)
            return carry

        lax.fori_loop(0, seq // 2, pair_step, 0)

        for s in (seq - 2, seq - 1):
            pltpu.make_async_copy(
                out_v.at[s % 2], out_hbm.at[s, h], osem.at[s % 2]
            ).wait()


def kernel(input_x, emb_table, pos_table):
    batch, seq_len = input_x.shape
    vocab, hid = emb_table.shape
    idx_t = input_x.T.astype(jnp.int32).reshape(seq_len, batch // 128, 128)
    emb_t = emb_table.T
    pos_t = pos_table[:seq_len].T

    mesh = plsc.VectorSubcoreMesh(
        core_axis_name="c", subcore_axis_name="s", num_cores=NC, num_subcores=NS
    )
    out = pl.kernel(
        _body,
        out_type=jax.ShapeDtypeStruct((seq_len, hid, batch), jnp.float32),
        mesh=mesh,
        scratch_types=[
            pltpu.VMEM((vocab,), jnp.float32),
            pltpu.VMEM((seq_len,), jnp.float32),
            pltpu.VMEM((2, batch // 128, 128), jnp.int32),
            pltpu.VMEM((2, batch), jnp.float32),
            pltpu.VMEM_SHARED((seq_len, batch // 128, 128), jnp.int32),
            pltpu.SemaphoreType.DMA((2,)),
            pltpu.SemaphoreType.DMA((2,)),
        ],
        compiler_params=pltpu.CompilerParams(
            use_tc_tiling_on_sc=True, needs_layout_passes=False
        ),
    )(idx_t, pos_t, emb_t)
    return out.transpose(2, 0, 1)


# speedup vs baseline: 3.4357x; 1.0871x over previous
import functools

import jax
import jax.numpy as jnp
from jax import lax
from jax.experimental import pallas as pl
from jax.experimental.pallas import tpu as pltpu
from jax.experimental.pallas import tpu_sc as plsc

NC = 2
NS = 16
NW = NC * NS

HID = 64
LANES = 16
JOBS = HID // NW


def _body(idx_hbm, pos_hbm, emb_hbm, out_hbm,
          row_v, pos_v, idx_v, out_v, idx_sh, isem, osem):
    seq = idx_hbm.shape[0]
    b_tot = idx_hbm.shape[1] * idx_hbm.shape[2]
    vocab = emb_hbm.shape[1]
    wid = lax.axis_index("s") * NC + lax.axis_index("c")

    sid = lax.axis_index("s")
    for k in range((seq + NS - 1) // NS):
        r = sid + k * NS

        @pl.when(r < seq)
        def _():
            for half in range(2):
                hr = idx_hbm.shape[1] // 2
                pltpu.sync_copy(idx_hbm.at[r, pl.ds(half * hr, hr)],
                                idx_v.at[half])
                pltpu.sync_copy(idx_v.at[half],
                                idx_sh.at[r, pl.ds(half * hr, hr)])

    plsc.subcore_barrier()

    NSLOT = 4
    HB = b_tot // 2
    HROW = HB // 128
    nsteps = 2 * seq

    def idx_src(t):
        return idx_sh.at[t // 2, pl.ds((t % 2) * HROW, HROW)]

    def start_idx_load(t, slot):
        pltpu.async_copy(idx_src(t), idx_v.at[slot], isem.at[slot])

    def wait_idx_load(t, slot):
        pltpu.make_async_copy(idx_src(t), idx_v.at[slot], isem.at[slot]).wait()

    def out_dst(t, h):
        return out_hbm.at[t // 2, h, pl.ds((t % 2) * HB, HB)]

    for j in range(JOBS):
        h = wid * JOBS + j
        pltpu.sync_copy(emb_hbm.at[h], row_v)
        pltpu.sync_copy(pos_hbm.at[h], pos_v)

        for u in range(NSLOT - 1):
            start_idx_load(u, u)

        def process(t, slot):
            @pl.when(t + (NSLOT - 1) < nsteps)
            def _():
                start_idx_load(t + (NSLOT - 1), (slot + NSLOT - 1) % NSLOT)

            wait_idx_load(t, slot)

            @pl.when(t >= NSLOT)
            def _():
                pltpu.make_async_copy(
                    out_v.at[slot], out_dst(t - NSLOT, h), osem.at[slot]
                ).wait()

            sv = jnp.full((LANES,), t // 2, jnp.int32)
            ps = plsc.load_gather(pos_v, [sv])

            GRP = 16

            def vec_step(g, carry2):
                t16s = [idx_v[slot, 2 * g + (k // 8), pl.ds((k % 8) * LANES, LANES)]
                        for k in range(GRP)]
                gots = [plsc.load_gather(row_v, [t16]) for t16 in t16s]
                for k in range(GRP):
                    out_v[slot, pl.ds(g * (GRP * LANES) + k * LANES, LANES)] = (
                        gots[k] + ps)
                return carry2

            lax.fori_loop(0, HB // (GRP * LANES), vec_step, 0)

            pltpu.async_copy(out_v.at[slot], out_dst(t, h), osem.at[slot])

        def quad_step(q, carry):
            for u in range(NSLOT):
                process(NSLOT * q + u, u)
            return carry

        lax.fori_loop(0, nsteps // NSLOT, quad_step, 0)

        for u in range(NSLOT):
            t = nsteps - NSLOT + u
            pltpu.make_async_copy(
                out_v.at[t % NSLOT], out_dst(t, h), osem.at[t % NSLOT]
            ).wait()


def kernel(input_x, emb_table, pos_table):
    batch, seq_len = input_x.shape
    vocab, hid = emb_table.shape
    idx_t = input_x.T.astype(jnp.int32).reshape(seq_len, batch // 128, 128)
    emb_t = emb_table.T
    pos_t = pos_table[:seq_len].T

    mesh = plsc.VectorSubcoreMesh(
        core_axis_name="c", subcore_axis_name="s", num_cores=NC, num_subcores=NS
    )
    out = pl.kernel(
        _body,
        out_type=jax.ShapeDtypeStruct((seq_len, hid, batch), jnp.float32),
        mesh=mesh,
        scratch_types=[
            pltpu.VMEM((vocab,), jnp.float32),
            pltpu.VMEM((seq_len,), jnp.float32),
            pltpu.VMEM((4, batch // 256, 128), jnp.int32),
            pltpu.VMEM((4, batch // 2), jnp.float32),
            pltpu.VMEM_SHARED((seq_len, batch // 128, 128), jnp.int32),
            pltpu.SemaphoreType.DMA((4,)),
            pltpu.SemaphoreType.DMA((4,)),
        ],
        compiler_params=pltpu.CompilerParams(
            use_tc_tiling_on_sc=True, needs_layout_passes=False
        ),
    )(idx_t, pos_t, emb_t)
    return out.transpose(2, 0, 1)


# speedup vs baseline: 3.5771x; 1.0412x over previous
import functools

import jax
import jax.numpy as jnp
from jax import lax
from jax.experimental import pallas as pl
from jax.experimental.pallas import tpu as pltpu
from jax.experimental.pallas import tpu_sc as plsc

NC = 2
NS = 16
NW = NC * NS

HID = 64
LANES = 16
JOBS = HID // NW


def _body(idx_hbm, pos_hbm, emb_hbm, out_hbm,
          row_v, pos_v, idx_v, out_v, idx_sh, isem, osem):
    seq = idx_hbm.shape[0]
    b_tot = idx_hbm.shape[1] * idx_hbm.shape[2]
    vocab = emb_hbm.shape[1]
    wid = lax.axis_index("s") * NC + lax.axis_index("c")

    sid = lax.axis_index("s")
    hr = idx_hbm.shape[1] // 2
    pairs = [(k, half)
             for k in range((seq + NS - 1) // NS) for half in range(2)]
    for r0 in range(0, len(pairs), 4):
        grp = pairs[r0:r0 + 4]
        for i, (k, half) in enumerate(grp):
            r = sid + k * NS

            @pl.when(r < seq)
            def _(r=r, half=half, i=i):
                pltpu.async_copy(idx_hbm.at[r, pl.ds(half * hr, hr)],
                                 idx_v.at[i], isem.at[i])

        for i, (k, half) in enumerate(grp):
            r = sid + k * NS

            @pl.when(r < seq)
            def _(r=r, half=half, i=i):
                pltpu.make_async_copy(idx_hbm.at[r, pl.ds(half * hr, hr)],
                                      idx_v.at[i], isem.at[i]).wait()
                pltpu.sync_copy(idx_v.at[i],
                                idx_sh.at[r, pl.ds(half * hr, hr)])

    plsc.subcore_barrier()

    NSLOT = 4
    HB = b_tot // 2
    HROW = HB // 128
    nsteps = 2 * seq

    def idx_src(t):
        return idx_sh.at[t // 2, pl.ds((t % 2) * HROW, HROW)]

    def start_idx_load(t, slot):
        pltpu.async_copy(idx_src(t), idx_v.at[slot], isem.at[slot])

    def wait_idx_load(t, slot):
        pltpu.make_async_copy(idx_src(t), idx_v.at[slot], isem.at[slot]).wait()

    def out_dst(t, h):
        return out_hbm.at[t // 2, h, pl.ds((t % 2) * HB, HB)]

    for j in range(JOBS):
        h = wid * JOBS + j
        pltpu.sync_copy(emb_hbm.at[h], row_v)
        pltpu.sync_copy(pos_hbm.at[h], pos_v)

        for u in range(NSLOT - 1):
            start_idx_load(u, u)

        def process(t, slot):
            @pl.when(t + (NSLOT - 1) < nsteps)
            def _():
                start_idx_load(t + (NSLOT - 1), (slot + NSLOT - 1) % NSLOT)

            wait_idx_load(t, slot)

            @pl.when(t >= NSLOT)
            def _():
                pltpu.make_async_copy(
                    out_v.at[slot], out_dst(t - NSLOT, h), osem.at[slot]
                ).wait()

            sv = jnp.full((LANES,), t // 2, jnp.int32)
            ps = plsc.load_gather(pos_v, [sv])

            GRP = 16

            def vec_step(g, carry2):
                t16s = [idx_v[slot, 2 * g + (k // 8), pl.ds((k % 8) * LANES, LANES)]
                        for k in range(GRP)]
                gots = [plsc.load_gather(row_v, [t16]) for t16 in t16s]
                for k in range(GRP):
                    out_v[slot, pl.ds(g * (GRP * LANES) + k * LANES, LANES)] = (
                        gots[k] + ps)
                return carry2

            lax.fori_loop(0, HB // (GRP * LANES), vec_step, 0)

            pltpu.async_copy(out_v.at[slot], out_dst(t, h), osem.at[slot])

        def quad_step(q, carry):
            for u in range(NSLOT):
                process(NSLOT * q + u, u)
            return carry

        lax.fori_loop(0, nsteps // NSLOT, quad_step, 0)

        for u in range(NSLOT):
            t = nsteps - NSLOT + u
            pltpu.make_async_copy(
                out_v.at[t % NSLOT], out_dst(t, h), osem.at[t % NSLOT]
            ).wait()


def kernel(input_x, emb_table, pos_table):
    batch, seq_len = input_x.shape
    vocab, hid = emb_table.shape
    idx_t = input_x.T.astype(jnp.int32).reshape(seq_len, batch // 128, 128)
    emb_t = emb_table.T
    pos_t = pos_table[:seq_len].T

    mesh = plsc.VectorSubcoreMesh(
        core_axis_name="c", subcore_axis_name="s", num_cores=NC, num_subcores=NS
    )
    out = pl.kernel(
        _body,
        out_type=jax.ShapeDtypeStruct((seq_len, hid, batch), jnp.float32),
        mesh=mesh,
        scratch_types=[
            pltpu.VMEM((vocab,), jnp.float32),
            pltpu.VMEM((seq_len,), jnp.float32),
            pltpu.VMEM((4, batch // 256, 128), jnp.int32),
            pltpu.VMEM((4, batch // 2), jnp.float32),
            pltpu.VMEM_SHARED((seq_len, batch // 128, 128), jnp.int32),
            pltpu.SemaphoreType.DMA((4,)),
            pltpu.SemaphoreType.DMA((4,)),
        ],
        compiler_params=pltpu.CompilerParams(
            use_tc_tiling_on_sc=True, needs_layout_passes=False
        ),
    )(idx_t, pos_t, emb_t)
    return out.transpose(2, 0, 1)
